# baseline (speedup 1.0000x reference)
import functools

import jax
import jax.numpy as jnp
from jax import lax
from jax.experimental import pallas as pl
from jax.experimental.pallas import tpu as pltpu
from jax.experimental.pallas import tpu_sc as plsc

B = 4096
V = 200
K = 128
D = 64

NUM_CORES = 2
NUM_SUBCORES = 16
NW = NUM_CORES * NUM_SUBCORES
BPW = B // NW


def _sc_gather(x2d, idx2d):
  mesh = plsc.VectorSubcoreMesh(core_axis_name="c", subcore_axis_name="s")

  @functools.partial(
      pl.kernel,
      mesh=mesh,
      out_type=jax.ShapeDtypeStruct((B * K, D), jnp.float32),
      compiler_params=pltpu.CompilerParams(use_tc_tiling_on_sc=False),
      scratch_types=[
          pltpu.VMEM((K,), jnp.int32),
          pltpu.VMEM((K, D), jnp.float32),
          pltpu.SemaphoreType.DMA,
      ],
  )
  def k(x_hbm, idx_hbm, out_hbm, idx_v, rows_v, sem):
    cid = lax.axis_index("c")
    sid = lax.axis_index("s")
    wid = sid * NUM_CORES + cid
    base = wid * BPW

    def body(b, carry):
      pltpu.sync_copy(idx_hbm.at[base + b], idx_v)
      pltpu.async_copy(x_hbm.at[idx_v], rows_v, sem).wait()
      pltpu.sync_copy(rows_v, out_hbm.at[pl.ds((base + b) * K, K)])
      return carry

    lax.fori_loop(0, BPW, body, 0)

  return k(x2d, idx2d)


def kernel(x, index):
  idx2d = (jnp.arange(B, dtype=jnp.int32)[:, None] * V
           + index.astype(jnp.int32)[None, :])
  x2d = x.reshape(B * V, D)
  out = _sc_gather(x2d, idx2d)
  return out.reshape(B, K, D)


# speedup vs baseline: 1.1856x; 1.1856x over previous
import functools

import jax
import jax.numpy as jnp
from jax import lax
from jax.experimental import pallas as pl
from jax.experimental.pallas import tpu as pltpu
from jax.experimental.pallas import tpu_sc as plsc

B = 4096
V = 200
K = 128
D = 64

NUM_CORES = 2
NUM_SUBCORES = 16
NW = NUM_CORES * NUM_SUBCORES
BPW = B // NW
CB = 4
NCH = BPW // CB


def _sc_gather(x2d, idx2d):
  mesh = plsc.VectorSubcoreMesh(core_axis_name="c", subcore_axis_name="s")

  @functools.partial(
      pl.kernel,
      mesh=mesh,
      out_type=jax.ShapeDtypeStruct((B * K, D), jnp.float32),
      compiler_params=pltpu.CompilerParams(use_tc_tiling_on_sc=False),
      scratch_types=[
          pltpu.VMEM((BPW, K), jnp.int32),
          pltpu.VMEM((CB * K, D), jnp.float32),
          pltpu.VMEM((CB * K, D), jnp.float32),
          pltpu.SemaphoreType.DMA,
          pltpu.SemaphoreType.DMA,
          pltpu.SemaphoreType.DMA,
          pltpu.SemaphoreType.DMA,
      ],
  )
  def k(x_hbm, idx_hbm, out_hbm, idx_v, rows0, rows1, g0, g1, s0, s1):
    cid = lax.axis_index("c")
    sid = lax.axis_index("s")
    wid = sid * NUM_CORES + cid
    base = wid * BPW

    pltpu.sync_copy(idx_hbm.at[pl.ds(base, BPW)], idx_v)

    def gathers(c, buf, sem):
      return [
          pltpu.make_async_copy(
              x_hbm.at[idx_v.at[c * CB + i]],
              buf.at[pl.ds(i * K, K)],
              sem,
          )
          for i in range(CB)
      ]

    def store(c, buf, sem):
      return pltpu.make_async_copy(
          buf, out_hbm.at[pl.ds((base + c * CB) * K, CB * K)], sem)

    def start_gathers(c, buf, sem):
      for cp in gathers(c, buf, sem):
        cp.start()

    def wait_gathers(c, buf, sem):
      for cp in gathers(c, buf, sem):
        cp.wait()

    start_gathers(0, rows0, g0)
    start_gathers(1, rows1, g1)

    def body(cc, carry):
      c0 = cc * 2
      c1 = c0 + 1

      wait_gathers(c0, rows0, g0)
      store(c0, rows0, s0).start()

      @pl.when(cc < NCH // 2 - 1)
      def _():
        store(c0, rows0, s0).wait()
        start_gathers(c0 + 2, rows0, g0)

      wait_gathers(c1, rows1, g1)
      store(c1, rows1, s1).start()

      @pl.when(cc < NCH // 2 - 1)
      def _():
        store(c1, rows1, s1).wait()
        start_gathers(c1 + 2, rows1, g1)

      return carry

    lax.fori_loop(0, NCH // 2, body, 0)

    store(NCH - 2, rows0, s0).wait()
    store(NCH - 1, rows1, s1).wait()

  return k(x2d, idx2d)


def kernel(x, index):
  idx2d = (jnp.arange(B, dtype=jnp.int32)[:, None] * V
           + index.astype(jnp.int32)[None, :])
  x2d = x.reshape(B * V, D)
  out = _sc_gather(x2d, idx2d)
  return out.reshape(B, K, D)


# speedup vs baseline: 4.9762x; 4.1973x over previous
import functools

import jax
import jax.numpy as jnp
from jax import lax
from jax.experimental import pallas as pl
from jax.experimental.pallas import tpu as pltpu
from jax.experimental.pallas import tpu_sc as plsc

B = 4096
V = 200
K = 128
D = 64

SLAB = B * D
S = 32
ROWS = SLAB // 128
CPS = ROWS // S
GSZ = 8

NUM_CORES = 2
NUM_SUBCORES = 16
NW = NUM_CORES * NUM_SUBCORES
SPW = K // NW
CPW = SPW * CPS
ITERS = CPW // GSZ


def _sc_slab_gather(xc, cidx):
  mesh = plsc.VectorSubcoreMesh(core_axis_name="c", subcore_axis_name="s")

  @functools.partial(
      pl.kernel,
      mesh=mesh,
      out_type=jax.ShapeDtypeStruct((K * CPS, S, 128), jnp.float32),
      scratch_types=[
          pltpu.VMEM((ITERS, GSZ), jnp.int32),
          pltpu.VMEM((GSZ, S, 128), jnp.float32),
          pltpu.VMEM((GSZ, S, 128), jnp.float32),
          pltpu.SemaphoreType.DMA,
          pltpu.SemaphoreType.DMA,
          pltpu.SemaphoreType.DMA,
          pltpu.SemaphoreType.DMA,
      ],
  )
  def k(x_hbm, cidx_hbm, out_hbm, idx_v, buf0, buf1, g0, g1, s0, s1):
    cid = lax.axis_index("c")
    sid = lax.axis_index("s")
    wid = sid * NUM_CORES + cid
    obase = wid * CPW

    pltpu.sync_copy(cidx_hbm.at[pl.ds(wid * ITERS, ITERS)], idx_v)

    def gather(i, buf, sem):
      return pltpu.make_async_copy(x_hbm.at[idx_v.at[i]], buf, sem)

    def store(i, buf, sem):
      return pltpu.make_async_copy(
          buf, out_hbm.at[pl.ds(obase + i * GSZ, GSZ)], sem)

    gather(0, buf0, g0).start()
    gather(1, buf1, g1).start()

    def body(cc, carry):
      i0 = cc * 2
      i1 = i0 + 1

      gather(i0, buf0, g0).wait()
      store(i0, buf0, s0).start()

      @pl.when(cc < ITERS // 2 - 1)
      def _():
        store(i0, buf0, s0).wait()
        gather(i0 + 2, buf0, g0).start()

      gather(i1, buf1, g1).wait()
      store(i1, buf1, s1).start()

      @pl.when(cc < ITERS // 2 - 1)
      def _():
        store(i1, buf1, s1).wait()
        gather(i1 + 2, buf1, g1).start()

      return carry

    lax.fori_loop(0, ITERS // 2, body, 0)

    store(ITERS - 2, buf0, s0).wait()
    store(ITERS - 1, buf1, s1).wait()

  return k(xc, cidx)


def kernel(x, index):
  xc = (x.transpose(1, 2, 0)
        .reshape(V, D // 8, 8, B // 128, 128)
        .transpose(0, 1, 3, 2, 4)
        .reshape(V * CPS, S, 128))
  cidx = (index.astype(jnp.int32)[:, None] * CPS
          + jnp.arange(CPS, dtype=jnp.int32)[None, :]).reshape(-1, GSZ)
  out = _sc_slab_gather(xc, cidx)
  return (out.reshape(K, D // 8, B // 128, 8, 128)
          .transpose(0, 1, 3, 2, 4)
          .reshape(K, D, B)
          .transpose(2, 0, 1))


# speedup vs baseline: 4.9856x; 1.0019x over previous
import functools

import jax
import jax.numpy as jnp
from jax import lax
from jax.experimental import pallas as pl
from jax.experimental.pallas import tpu as pltpu
from jax.experimental.pallas import tpu_sc as plsc

B = 4096
V = 200
K = 128
D = 64

SLAB = B * D
S = 32
ROWS = SLAB // 128
CPS = ROWS // S
GSZ = 4

NUM_CORES = 2
NUM_SUBCORES = 16
NW = NUM_CORES * NUM_SUBCORES
SPW = K // NW
CPW = SPW * CPS
ITERS = CPW // GSZ
NBUF = 4


def _sc_slab_gather(xc, cidx):
  mesh = plsc.VectorSubcoreMesh(core_axis_name="c", subcore_axis_name="s")

  @functools.partial(
      pl.kernel,
      mesh=mesh,
      out_type=jax.ShapeDtypeStruct((K * CPS, S, 128), jnp.float32),
      scratch_types=[
          pltpu.VMEM((ITERS, GSZ), jnp.int32),
          [pltpu.VMEM((GSZ, S, 128), jnp.float32) for _ in range(NBUF)],
          [pltpu.SemaphoreType.DMA for _ in range(NBUF)],
          [pltpu.SemaphoreType.DMA for _ in range(NBUF)],
      ],
  )
  def k(x_hbm, cidx_hbm, out_hbm, idx_v, bufs, gsems, ssems):
    cid = lax.axis_index("c")
    sid = lax.axis_index("s")
    wid = sid * NUM_CORES + cid
    obase = wid * CPW

    pltpu.sync_copy(cidx_hbm.at[pl.ds(wid * ITERS, ITERS)], idx_v)

    def gather(i, b):
      return pltpu.make_async_copy(x_hbm.at[idx_v.at[i]], bufs[b], gsems[b])

    def store(i, b):
      return pltpu.make_async_copy(
          bufs[b], out_hbm.at[pl.ds(obase + i * GSZ, GSZ)], ssems[b])

    for b in range(NBUF):
      gather(b, b).start()

    def body(cc, carry):
      for kk in range(NBUF):
        i = cc * NBUF + kk
        gather(i, kk).wait()
        store(i, kk).start()

        @pl.when((i >= 1) & (i + NBUF - 1 < ITERS))
        def _():
          pb = (kk + NBUF - 1) % NBUF
          store(i - 1, pb).wait()
          gather(i + NBUF - 1, pb).start()

      return carry

    lax.fori_loop(0, ITERS // NBUF, body, 0)

    for t in range(NBUF):
      i = ITERS - NBUF + t
      store(i, i % NBUF).wait()

  return k(xc, cidx)


def kernel(x, index):
  xc = (x.transpose(1, 2, 0)
        .reshape(V, D // 8, 8, B // 128, 128)
        .transpose(0, 1, 3, 2, 4)
        .reshape(V * CPS, S, 128))
  cidx = (index.astype(jnp.int32)[:, None] * CPS
          + jnp.arange(CPS, dtype=jnp.int32)[None, :]).reshape(-1, GSZ)
  out = _sc_slab_gather(xc, cidx)
  return (out.reshape(K, D // 8, B // 128, 8, 128)
          .transpose(0, 1, 3, 2, 4)
          .reshape(K, D, B)
          .transpose(2, 0, 1))
